# baseline (speedup 1.0000x reference)
import functools

import jax
import jax.numpy as jnp
from jax import lax
from jax.experimental import pallas as pl
from jax.experimental.pallas import tpu as pltpu
from jax.experimental.pallas import tpu_sc as plsc

NV = 2
NN = 10000
EE = 320000
FT = 128
FH = 64
NPAD = 10240
NC = 2
NS = 16
ROWS_PER_TILE = NPAD // NS
E_PER_TILE = EE // NS
CHUNK = 80
NCHUNK = E_PER_TILE // CHUNK
NGRP = 10
GRP = NCHUNK // NGRP
DEPTH = 5
DEG_R0 = NN
DROWS = 160


def _sc_segment_sum(x_lo, x_hi, src_r, dst_r):
  mesh = plsc.VectorSubcoreMesh(
      core_axis_name="c", subcore_axis_name="s", num_cores=NC,
      num_subcores=NS)

  @functools.partial(
      pl.kernel,
      out_type=jax.ShapeDtypeStruct((NV, 2, NPAD, FH), jnp.float32),
      mesh=mesh,
      scratch_types=[
          pltpu.VMEM_SHARED((NPAD, FH), jnp.float32),
          pltpu.VMEM((DEPTH, CHUNK, FH), jnp.float32),
          pltpu.VMEM((GRP, CHUNK), jnp.int32),
          pltpu.VMEM((GRP, CHUNK), jnp.int32),
          pltpu.VMEM((DROWS, FH), jnp.float32),
          [pltpu.SemaphoreType.DMA] * DEPTH,
          [pltpu.SemaphoreType.DMA] * DEPTH,
      ],
      compiler_params=pltpu.CompilerParams(needs_layout_passes=False,
                                           use_tc_tiling_on_sc=False),
  )
  def k(xl_hbm, xh_hbm, src_hbm, dst_hbm, agg_out,
        agg_s, rows, srcbuf, dstbuf, degloc, gsems, ssems):
    c = lax.axis_index("c")
    s = lax.axis_index("s")
    zeros16 = jnp.zeros((16,), jnp.float32)
    ones16 = jnp.full((16,), 1.0, jnp.float32)
    base = s * ROWS_PER_TILE

    def zr(r, carry):
      def zc(j, c2):
        sl = pl.ds(j * 16, 16)
        rows[0, r, sl] = zeros16
        degloc[r, sl] = zeros16
        degloc[CHUNK + r, sl] = zeros16
        return c2
      lax.fori_loop(0, FH // 16, zc, 0)
      return carry
    lax.fori_loop(0, CHUNK, zr, 0)

    def hist(kk):
      for j in range(CHUNK // 16):
        idx16 = dstbuf[kk, pl.ds(j * 16, 16)]
        row16 = lax.shift_right_logical(idx16, 6)
        col16 = lax.bitwise_and(idx16, 63)
        plsc.addupdate_scatter(degloc, [row16, col16], ones16)

    for p in range(2):
      x_hbm = xl_hbm if p == 0 else xh_hbm

      def gather(kk, b, sem):
        pltpu.async_copy(x_hbm.at[srcbuf.at[kk]], rows.at[b], sem)

      def wait_gather(kk, b, sem):
        pltpu.make_async_copy(
            x_hbm.at[srcbuf.at[kk]], rows.at[b], sem).wait()

      def scatter(kk, b, sem):
        pltpu.async_copy(rows.at[b], agg_s.at[dstbuf.at[kk]], sem,
                         add=True)

      def wait_scatter(kk, b, sem):
        pltpu.make_async_copy(
            rows.at[b], agg_s.at[dstbuf.at[kk]], sem).wait()

      def za(t, carry):
        pltpu.sync_copy(rows.at[0],
                        agg_s.at[pl.ds(base + t * CHUNK, CHUNK)])
        return carry
      lax.fori_loop(0, ROWS_PER_TILE // CHUNK, za, 0)

      plsc.subcore_barrier()

      def grp(g, carry):
        pltpu.sync_copy(src_hbm.at[c, s, g], srcbuf)
        pltpu.sync_copy(dst_hbm.at[c, s, g], dstbuf)
        for b in range(DEPTH):
          gather(b, b, gsems[b])

        def stage(t, c2):
          k0 = DEPTH * t
          for b in range(DEPTH):
            wait_gather(k0 + b, b, gsems[b])
            scatter(k0 + b, b, ssems[b])
            if p == 0:
              hist(k0 + b)
          for b in range(DEPTH):
            wait_scatter(k0 + b, b, ssems[b])
            gather(k0 + DEPTH + b, b, gsems[b])
          return c2
        lax.fori_loop(0, GRP // DEPTH - 1, stage, 0)

        k0 = GRP - DEPTH
        for b in range(DEPTH):
          wait_gather(k0 + b, b, gsems[b])
          scatter(k0 + b, b, ssems[b])
          if p == 0:
            hist(k0 + b)
        for b in range(DEPTH):
          wait_scatter(k0 + b, b, ssems[b])
        return carry
      lax.fori_loop(0, NGRP, grp, 0)

      if p == 0:
        def it(j, carry):
          io16 = lax.broadcasted_iota(jnp.int32, (16,), 0)
          dstbuf[0, pl.ds(j * 16, 16)] = io16 + (DEG_R0 + j * 16)
          dstbuf[1, pl.ds(j * 16, 16)] = io16 + (DEG_R0 + CHUNK + j * 16)
          return carry
        lax.fori_loop(0, CHUNK // 16, it, 0)
        pltpu.sync_copy(degloc.at[pl.ds(0, CHUNK)],
                        agg_s.at[dstbuf.at[0]], add=True)
        pltpu.sync_copy(degloc.at[pl.ds(CHUNK, CHUNK)],
                        agg_s.at[dstbuf.at[1]], add=True)
      plsc.subcore_barrier()

      pltpu.sync_copy(agg_s.at[pl.ds(base, ROWS_PER_TILE)],
                      agg_out.at[c, p, pl.ds(base, ROWS_PER_TILE)])
      if p == 0:
        def rz(r, carry):
          def zc(j, c2):
            rows[0, r, pl.ds(j * 16, 16)] = zeros16
            return c2
          lax.fori_loop(0, FH // 16, zc, 0)
          return carry
        lax.fori_loop(0, CHUNK, rz, 0)

  return k(x_lo, x_hi, src_r, dst_r)


def _tc_body(lo_ref, hi_ref, deg_ref, wg, bg, wsp, bsp, w1, b1, w2, b2,
             wo, bo, c_out, p_out, r_out):
  a = jnp.concatenate([lo_ref[0, 0], hi_ref[0, 0]], axis=1)
  d = deg_ref[0]
  h = a / jnp.maximum(d, 1.0)
  h = jnp.maximum(jnp.dot(h, wg[0], preferred_element_type=jnp.float32)
                  + bg[0], 0.0)
  z = jnp.dot(h, wsp[0], preferred_element_type=jnp.float32) + bsp[0]
  c_out[0] = z[:, :FH]
  p_out[0] = z[:, FH:]
  dd = jnp.maximum(jnp.dot(z, w1[0], preferred_element_type=jnp.float32)
                   + b1[0], 0.0)
  e = jnp.dot(dd, w2[0], preferred_element_type=jnp.float32) + b2[0]
  r = jnp.dot(jnp.maximum(e, 0.0), wo[0],
              preferred_element_type=jnp.float32) + bo[0]
  r_out[0] = r


def _tc_dense(agg, deg3, W_gcn, b_gcn, W_sp, b_sp, W_d1, b_d1, W_d2, b_d2,
              W_out, b_out):
  BN = 2000
  nb = NN // BN
  wspec = pl.BlockSpec((1, FT, FT), lambda v, b: (v, 0, 0))
  bspec = pl.BlockSpec((1, 1, FT), lambda v, b: (v, 0, 0))
  return pl.pallas_call(
      _tc_body,
      grid=(NV, nb),
      in_specs=[
          pl.BlockSpec((1, 1, BN, FH), lambda v, b: (v, 0, b, 0)),
          pl.BlockSpec((1, 1, BN, FH), lambda v, b: (v, 1, b, 0)),
          pl.BlockSpec((1, BN, 1), lambda v, b: (v, b, 0)),
          wspec, bspec, wspec, bspec, wspec, bspec, wspec, bspec,
          wspec, bspec,
      ],
      out_specs=[
          pl.BlockSpec((1, BN, FH), lambda v, b: (v, b, 0)),
          pl.BlockSpec((1, BN, FH), lambda v, b: (v, b, 0)),
          pl.BlockSpec((1, BN, FT), lambda v, b: (v, b, 0)),
      ],
      out_shape=[
          jax.ShapeDtypeStruct((NV, NN, FH), jnp.float32),
          jax.ShapeDtypeStruct((NV, NN, FH), jnp.float32),
          jax.ShapeDtypeStruct((NV, NN, FT), jnp.float32),
      ],
  )(agg, agg, deg3, W_gcn, b_gcn, W_sp, b_sp, W_d1, b_d1, W_d2, b_d2,
    W_out, b_out)


def kernel(x, adj, W_gcn, b_gcn, W_s, b_s, W_p, b_p, W_d1, b_d1, W_d2, b_d2,
           W_out, b_out):
  xflat = x.reshape(NV * NN, FT)
  x_lo = xflat[:, :FH]
  x_hi = xflat[:, FH:]
  view_off = (jnp.arange(NV, dtype=jnp.int32) * NN)[:, None]
  src_r = (adj[:, 0, :] + view_off).reshape(NV, NS, NGRP, GRP, CHUNK)
  dst_r = adj[:, 1, :].reshape(NV, NS, NGRP, GRP, CHUNK)

  agg = _sc_segment_sum(x_lo, x_hi, src_r, dst_r)

  deg3 = agg[:, 0, DEG_R0:DEG_R0 + DROWS, :].reshape(NV, DROWS * FH)
  deg3 = deg3[:, :NPAD, None]

  W_sp = jnp.concatenate([W_s, W_p], axis=2)
  b_sp = jnp.concatenate([b_s, b_p], axis=1)
  commons, privates, recons = _tc_dense(
      agg, deg3, W_gcn, b_gcn[:, None, :], W_sp, b_sp[:, None, :],
      W_d1, b_d1[:, None, :], W_d2, b_d2[:, None, :], W_out,
      b_out[:, None, :])
  return (commons, privates, recons)


# speedup vs baseline: 1.0914x; 1.0914x over previous
import functools

import jax
import jax.numpy as jnp
from jax import lax
from jax.experimental import pallas as pl
from jax.experimental.pallas import tpu as pltpu
from jax.experimental.pallas import tpu_sc as plsc

NV = 2
NN = 10000
EE = 320000
FT = 128
FH = 64
NPAD = 10112
NC = 2
NS = 16
ROWS_PER_TILE = NPAD // NS
E_PER_TILE = EE // NS
CHUNK = 80
NCHUNK = E_PER_TILE // CHUNK
NGRP = 10
GRP = NCHUNK // NGRP
DEG_R0 = NN
DROWS = 80


def _sc_segment_sum(xflat, src_r, dst_r):
  mesh = plsc.VectorSubcoreMesh(
      core_axis_name="c", subcore_axis_name="s", num_cores=NC,
      num_subcores=NS)

  @functools.partial(
      pl.kernel,
      out_type=jax.ShapeDtypeStruct((NV, NPAD, FT), jnp.float32),
      mesh=mesh,
      scratch_types=[
          pltpu.VMEM_SHARED((NPAD, FT), jnp.float32),
          pltpu.VMEM((CHUNK, FT), jnp.float32),
          pltpu.VMEM((CHUNK, FT), jnp.float32),
          pltpu.VMEM((CHUNK, FT), jnp.float32),
          pltpu.VMEM((GRP, CHUNK), jnp.int32),
          pltpu.VMEM((GRP, CHUNK), jnp.int32),
          pltpu.VMEM((DROWS, 128), jnp.float32),
          pltpu.SemaphoreType.DMA,
          pltpu.SemaphoreType.DMA,
          pltpu.SemaphoreType.DMA,
          pltpu.SemaphoreType.DMA,
          pltpu.SemaphoreType.DMA,
          pltpu.SemaphoreType.DMA,
      ],
      compiler_params=pltpu.CompilerParams(needs_layout_passes=False),
  )
  def k(x_hbm, src_hbm, dst_hbm, agg_out,
        agg_s, rows0, rows1, rows2, srcbuf, dstbuf, degloc,
        gsem0, gsem1, gsem2, ssem0, ssem1, ssem2):
    c = lax.axis_index("c")
    s = lax.axis_index("s")
    zeros16 = jnp.zeros((16,), jnp.float32)
    ones16 = jnp.full((16,), 1.0, jnp.float32)

    def zr(r, carry):
      def zc(j, c2):
        sl = pl.ds(j * 16, 16)
        rows0[r, sl] = zeros16
        degloc[r, sl] = zeros16
        return c2
      lax.fori_loop(0, FT // 16, zc, 0)
      return carry
    lax.fori_loop(0, CHUNK, zr, 0)

    base = s * ROWS_PER_TILE
    def za(t, carry):
      pltpu.sync_copy(rows0, agg_s.at[pl.ds(base + t * CHUNK, CHUNK)])
      return carry
    lax.fori_loop(0, 7, za, 0)
    pltpu.sync_copy(rows0.at[pl.ds(0, 72)],
                    agg_s.at[pl.ds(base + 7 * CHUNK, 72)])

    plsc.subcore_barrier()

    def hist(kk):
      for j in range(CHUNK // 16):
        idx16 = dstbuf[kk, pl.ds(j * 16, 16)]
        row16 = lax.shift_right_logical(idx16, 7)
        col16 = lax.bitwise_and(idx16, 127)
        plsc.addupdate_scatter(degloc, [row16, col16], ones16)

    def gather(kk, rows, gsem):
      pltpu.async_copy(x_hbm.at[srcbuf.at[kk]], rows, gsem)

    def wait_gather(kk, rows, gsem):
      pltpu.make_async_copy(x_hbm.at[srcbuf.at[kk]], rows, gsem).wait()

    def scatter(kk, rows, ssem):
      pltpu.async_copy(rows, agg_s.at[dstbuf.at[kk]], ssem, add=True)

    def wait_scatter(kk, rows, ssem):
      pltpu.make_async_copy(rows, agg_s.at[dstbuf.at[kk]], ssem).wait()

    def grp(g, carry):
      pltpu.sync_copy(src_hbm.at[c, s, g], srcbuf)
      pltpu.sync_copy(dst_hbm.at[c, s, g], dstbuf)
      gather(0, rows0, gsem0)
      gather(1, rows1, gsem1)
      gather(2, rows2, gsem2)

      def tri(t, c2):
        k0 = 3 * t
        wait_gather(k0, rows0, gsem0)
        scatter(k0, rows0, ssem0)
        hist(k0)
        wait_gather(k0 + 1, rows1, gsem1)
        scatter(k0 + 1, rows1, ssem1)
        hist(k0 + 1)
        wait_gather(k0 + 2, rows2, gsem2)
        scatter(k0 + 2, rows2, ssem2)
        hist(k0 + 2)
        wait_scatter(k0, rows0, ssem0)
        gather(k0 + 3, rows0, gsem0)
        wait_scatter(k0 + 1, rows1, ssem1)
        gather(k0 + 4, rows1, gsem1)
        wait_scatter(k0 + 2, rows2, ssem2)
        gather(k0 + 5, rows2, gsem2)
        return c2
      lax.fori_loop(0, 7, tri, 0)

      wait_gather(21, rows0, gsem0)
      scatter(21, rows0, ssem0)
      hist(21)
      wait_gather(22, rows1, gsem1)
      scatter(22, rows1, ssem1)
      hist(22)
      wait_gather(23, rows2, gsem2)
      scatter(23, rows2, ssem2)
      hist(23)
      wait_scatter(21, rows0, ssem0)
      gather(24, rows0, gsem0)
      wait_gather(24, rows0, gsem0)
      scatter(24, rows0, ssem0)
      hist(24)
      wait_scatter(22, rows1, ssem1)
      wait_scatter(23, rows2, ssem2)
      wait_scatter(24, rows0, ssem0)
      return carry
    lax.fori_loop(0, NGRP, grp, 0)

    def it(j, carry):
      dstbuf[0, pl.ds(j * 16, 16)] = (
          lax.broadcasted_iota(jnp.int32, (16,), 0) + (DEG_R0 + j * 16))
      return carry
    lax.fori_loop(0, DROWS // 16, it, 0)
    pltpu.sync_copy(degloc, agg_s.at[dstbuf.at[0]], add=True)
    plsc.subcore_barrier()

    pltpu.sync_copy(agg_s.at[pl.ds(base, ROWS_PER_TILE)],
                    agg_out.at[c, pl.ds(base, ROWS_PER_TILE)])

  return k(xflat, src_r, dst_r)


def _tc_body(agg_ref, deg_ref, wg, bg, wsp, bsp, w1, b1, w2, b2, wo, bo,
             z_out, r_out):
  a = agg_ref[0]
  d = deg_ref[0]
  h = a / jnp.maximum(d, 1.0)
  h = jnp.maximum(jnp.dot(h, wg[0], preferred_element_type=jnp.float32)
                  + bg[0], 0.0)
  z = jnp.dot(h, wsp[0], preferred_element_type=jnp.float32) + bsp[0]
  z_out[0] = z
  dd = jnp.maximum(jnp.dot(z, w1[0], preferred_element_type=jnp.float32)
                   + b1[0], 0.0)
  e = jnp.dot(dd, w2[0], preferred_element_type=jnp.float32) + b2[0]
  r = jnp.dot(jnp.maximum(e, 0.0), wo[0],
              preferred_element_type=jnp.float32) + bo[0]
  r_out[0] = r


def _tc_dense(agg, deg3, W_gcn, b_gcn, W_sp, b_sp, W_d1, b_d1, W_d2, b_d2,
              W_out, b_out):
  BN = NPAD // 8
  wspec = pl.BlockSpec((1, FT, FT), lambda v, b: (v, 0, 0))
  bspec = pl.BlockSpec((1, 1, FT), lambda v, b: (v, 0, 0))
  return pl.pallas_call(
      _tc_body,
      grid=(NV, 8),
      in_specs=[
          pl.BlockSpec((1, BN, FT), lambda v, b: (v, b, 0)),
          pl.BlockSpec((1, BN, 1), lambda v, b: (v, b, 0)),
          wspec, bspec, wspec, bspec, wspec, bspec, wspec, bspec,
          wspec, bspec,
      ],
      out_specs=[
          pl.BlockSpec((1, BN, FT), lambda v, b: (v, b, 0)),
          pl.BlockSpec((1, BN, FT), lambda v, b: (v, b, 0)),
      ],
      out_shape=[
          jax.ShapeDtypeStruct((NV, NPAD, FT), jnp.float32),
          jax.ShapeDtypeStruct((NV, NPAD, FT), jnp.float32),
      ],
  )(agg, deg3, W_gcn, b_gcn, W_sp, b_sp, W_d1, b_d1, W_d2, b_d2, W_out,
    b_out)


def kernel(x, adj, W_gcn, b_gcn, W_s, b_s, W_p, b_p, W_d1, b_d1, W_d2, b_d2,
           W_out, b_out):
  xflat = x.reshape(NV * NN, FT)
  view_off = (jnp.arange(NV, dtype=jnp.int32) * NN)[:, None]
  src_r = (adj[:, 0, :] + view_off).reshape(NV, NS, NGRP, GRP, CHUNK)
  dst_r = adj[:, 1, :].reshape(NV, NS, NGRP, GRP, CHUNK)

  agg = _sc_segment_sum(xflat, src_r, dst_r)

  deg = agg[:, DEG_R0:DEG_R0 + DROWS, :].reshape(NV, DROWS * 128)
  deg3 = deg[:, :NPAD, None]

  W_sp = jnp.concatenate([W_s, W_p], axis=2)
  b_sp = jnp.concatenate([b_s, b_p], axis=1)
  z, r = _tc_dense(agg, deg3, W_gcn, b_gcn[:, None, :], W_sp,
                   b_sp[:, None, :], W_d1, b_d1[:, None, :], W_d2,
                   b_d2[:, None, :], W_out, b_out[:, None, :])
  commons = z[:, :NN, :64]
  privates = z[:, :NN, 64:]
  recons = r[:, :NN, :]
  return (commons, privates, recons)
